# Initial kernel scaffold; baseline (speedup 1.0000x reference)
import jax
import jax.numpy as jnp
from jax.experimental import pallas as pl


def kernel(query, prompt_keys, prompt_values):
    raise NotImplementedError("write your pallas kernel here")


# speedup vs baseline: 1.4035x; 1.4035x over previous
import functools

import jax
import jax.numpy as jnp
from jax import lax
from jax.experimental import pallas as pl
from jax.experimental.pallas import tpu as pltpu
from jax.experimental.pallas import tpu_sc as plsc

POOL = 512
K = 4
LEN = 8
DIM = 768
BATCH = 1024
ROW = LEN * DIM

NUM_WORKERS = 32
ROWS_TOTAL = BATCH * K
ROWS_PER_W = ROWS_TOTAL // NUM_WORKERS
CHUNK = 16
NCHUNK = ROWS_PER_W // CHUNK


def _topk_body(q_ref, kt_ref, idx_ref, loss_ref):
    q = q_ref[...]
    kt = kt_ref[...]
    qn = q / jnp.maximum(jnp.sqrt(jnp.sum(q * q, axis=1, keepdims=True)), 1e-12)
    kn = kt / jnp.maximum(jnp.sqrt(jnp.sum(kt * kt, axis=0, keepdims=True)), 1e-12)
    d = jnp.dot(qn, kn, preferred_element_type=jnp.float32)
    lane = lax.broadcasted_iota(jnp.int32, d.shape, 1)
    loss = jnp.float32(0.0)
    for t in range(K):
        m = jnp.max(d, axis=1, keepdims=True)
        im = jnp.min(jnp.where(d == m, lane, POOL), axis=1, keepdims=True)
        idx_ref[:, t : t + 1] = im
        loss = loss + jnp.sum(jnp.abs(m))
        d = jnp.where(lane == im, -jnp.inf, d)
    loss_ref[...] = jnp.full((1, 1), loss / jnp.float32(BATCH), jnp.float32)


def _tc_topk(query, keys_t):
    return pl.pallas_call(
        _topk_body,
        out_shape=(
            jax.ShapeDtypeStruct((BATCH, K), jnp.int32),
            jax.ShapeDtypeStruct((1, 1), jnp.float32),
        ),
    )(query, keys_t)


def _sc_gather_body(idx_hbm, table_hbm, out_hbm, idx_v, rows_v, sem):
    wid = lax.axis_index("s") * 2 + lax.axis_index("c")
    base = wid * ROWS_PER_W
    for c in range(NCHUNK):
        off = base + c * CHUNK
        pltpu.sync_copy(idx_hbm.at[pl.ds(off, CHUNK)], idx_v)
        pltpu.async_copy(table_hbm.at[idx_v], rows_v, sem).wait()
        pltpu.sync_copy(rows_v, out_hbm.at[pl.ds(off, CHUNK)])


@functools.cache
def _sc_gather():
    return pl.kernel(
        _sc_gather_body,
        mesh=plsc.VectorSubcoreMesh(core_axis_name="c", subcore_axis_name="s"),
        out_type=jax.ShapeDtypeStruct((ROWS_TOTAL, ROW), jnp.float32),
        scratch_types=[
            pltpu.VMEM((CHUNK,), jnp.int32),
            pltpu.VMEM((CHUNK, ROW), jnp.float32),
            pltpu.SemaphoreType.DMA,
        ],
    )


def kernel(query, prompt_keys, prompt_values):
    keys_t = prompt_keys.T
    idx, loss = _tc_topk(query, keys_t)
    table = prompt_values.reshape(POOL, ROW)
    rows = _sc_gather()(idx.reshape(ROWS_TOTAL), table)
    return rows.reshape(BATCH, K, LEN, DIM), loss.reshape(())


# speedup vs baseline: 1.4505x; 1.0335x over previous
import functools

import jax
import jax.numpy as jnp
from jax import lax
from jax.experimental import pallas as pl
from jax.experimental.pallas import tpu as pltpu
from jax.experimental.pallas import tpu_sc as plsc

POOL = 512
K = 4
LEN = 8
DIM = 768
BATCH = 1024
ROW = LEN * DIM

NUM_WORKERS = 32
ROWS_TOTAL = BATCH * K
ROWS_PER_W = ROWS_TOTAL // NUM_WORKERS
CHUNK = 8
NCHUNK = ROWS_PER_W // CHUNK


def _topk_body(q_ref, k_ref, idx_ref, loss_ref):
    q = q_ref[...]
    k = k_ref[...]
    qn = q / jnp.maximum(jnp.sqrt(jnp.sum(q * q, axis=1, keepdims=True)), 1e-12)
    kn = k / jnp.maximum(jnp.sqrt(jnp.sum(k * k, axis=1, keepdims=True)), 1e-12)
    d = lax.dot_general(qn, kn, (((1,), (1,)), ((), ())),
                        preferred_element_type=jnp.float32)
    lane = lax.broadcasted_iota(jnp.int32, d.shape, 1)
    loss = jnp.float32(0.0)
    for t in range(K):
        m = jnp.max(d, axis=1, keepdims=True)
        im = jnp.min(jnp.where(d == m, lane, POOL), axis=1, keepdims=True)
        idx_ref[:, t : t + 1] = im
        loss = loss + jnp.sum(jnp.abs(m))
        d = jnp.where(lane == im, -jnp.inf, d)
    loss_ref[...] = jnp.full((1, 1), loss / jnp.float32(BATCH), jnp.float32)


def _tc_topk(query, keys_t):
    return pl.pallas_call(
        _topk_body,
        out_shape=(
            jax.ShapeDtypeStruct((BATCH, K), jnp.int32),
            jax.ShapeDtypeStruct((1, 1), jnp.float32),
        ),
    )(query, keys_t)


def _sc_gather_body(idx_hbm, table_hbm, out_hbm,
                    idx_v, rows_v0, rows_v1,
                    gsem0, gsem1, ssem0, ssem1):
    wid = lax.axis_index("s") * 2 + lax.axis_index("c")
    base = wid * ROWS_PER_W
    rows_v = (rows_v0, rows_v1)
    gsem = (gsem0, gsem1)
    ssem = (ssem0, ssem1)

    pltpu.sync_copy(idx_hbm.at[pl.ds(base, ROWS_PER_W)], idx_v)

    def fire_gather(c):
        pltpu.make_async_copy(
            table_hbm.at[idx_v.at[pl.ds(c * CHUNK, CHUNK)]],
            rows_v[c % 2], gsem[c % 2]).start()

    fire_gather(0)
    for c in range(NCHUNK):
        b = c % 2
        if c + 1 < NCHUNK:
            if c >= 1:
                pltpu.make_async_copy(
                    rows_v[1 - b], out_hbm.at[pl.ds(base + (c - 1) * CHUNK, CHUNK)],
                    ssem[1 - b]).wait()
            fire_gather(c + 1)
        pltpu.make_async_copy(
            table_hbm.at[idx_v.at[pl.ds(c * CHUNK, CHUNK)]],
            rows_v[b], gsem[b]).wait()
        pltpu.make_async_copy(
            rows_v[b], out_hbm.at[pl.ds(base + c * CHUNK, CHUNK)], ssem[b]).start()
    for c in (NCHUNK - 2, NCHUNK - 1):
        pltpu.make_async_copy(
            rows_v[c % 2], out_hbm.at[pl.ds(base + c * CHUNK, CHUNK)],
            ssem[c % 2]).wait()


@functools.cache
def _sc_gather():
    return pl.kernel(
        _sc_gather_body,
        mesh=plsc.VectorSubcoreMesh(core_axis_name="c", subcore_axis_name="s"),
        out_type=jax.ShapeDtypeStruct((ROWS_TOTAL, ROW), jnp.float32),
        scratch_types=[
            pltpu.VMEM((ROWS_PER_W,), jnp.int32),
            pltpu.VMEM((CHUNK, ROW), jnp.float32),
            pltpu.VMEM((CHUNK, ROW), jnp.float32),
            pltpu.SemaphoreType.DMA,
            pltpu.SemaphoreType.DMA,
            pltpu.SemaphoreType.DMA,
            pltpu.SemaphoreType.DMA,
        ],
    )


def kernel(query, prompt_keys, prompt_values):
    idx, loss = _tc_topk(query, prompt_keys)
    table = prompt_values.reshape(POOL, ROW)
    rows = _sc_gather()(idx.reshape(ROWS_TOTAL), table)
    return rows.reshape(BATCH, K, LEN, DIM), loss.reshape(())


# speedup vs baseline: 3.1108x; 2.1446x over previous
import functools

import jax
import jax.numpy as jnp
from jax import lax
from jax.experimental import pallas as pl
from jax.experimental.pallas import tpu as pltpu
from jax.experimental.pallas import tpu_sc as plsc

POOL = 512
K = 4
LEN = 8
DIM = 768
BATCH = 1024
ROW = LEN * DIM

NUM_WORKERS = 32
ROWS_TOTAL = BATCH * K
ROWS_PER_W = ROWS_TOTAL // NUM_WORKERS
CHUNK = 8
NCHUNK = ROWS_PER_W // CHUNK


def _topk_body(q_ref, k_ref, idx_ref, loss_ref):
    q = q_ref[...]
    k = k_ref[...]
    qn = q / jnp.maximum(jnp.sqrt(jnp.sum(q * q, axis=1, keepdims=True)), 1e-12)
    kn = k / jnp.maximum(jnp.sqrt(jnp.sum(k * k, axis=1, keepdims=True)), 1e-12)
    d = lax.dot_general(qn, kn, (((1,), (1,)), ((), ())),
                        preferred_element_type=jnp.float32)
    lane = lax.broadcasted_iota(jnp.int32, d.shape, 1)
    loss = jnp.float32(0.0)
    for t in range(K):
        m = jnp.max(d, axis=1, keepdims=True)
        im = jnp.min(jnp.where(d == m, lane, POOL), axis=1, keepdims=True)
        idx_ref[:, t : t + 1] = im
        loss = loss + jnp.sum(jnp.abs(m))
        d = jnp.where(lane == im, -jnp.inf, d)
    loss_ref[...] = jnp.full((1, 1), loss / jnp.float32(BATCH), jnp.float32)


def _tc_topk(query, keys_t):
    return pl.pallas_call(
        _topk_body,
        out_shape=(
            jax.ShapeDtypeStruct((BATCH, K), jnp.int32),
            jax.ShapeDtypeStruct((1, 1), jnp.float32),
        ),
    )(query, keys_t)


def _sc_gather_body(idx_hbm, table_hbm, out_hbm,
                    idx_v, rows_v0, rows_v1,
                    gsem0, gsem1, ssem0, ssem1):
    wid = lax.axis_index("s") * 2 + lax.axis_index("c")
    base = wid * ROWS_PER_W
    rows_v = (rows_v0, rows_v1)
    gsem = (gsem0, gsem1)
    ssem = (ssem0, ssem1)

    pltpu.sync_copy(idx_hbm.at[wid], idx_v)

    def fire_gather(c):
        pltpu.make_async_copy(
            table_hbm.at[idx_v.at[pl.ds(c * CHUNK, CHUNK)]],
            rows_v[c % 2], gsem[c % 2]).start()

    fire_gather(0)
    for c in range(NCHUNK):
        b = c % 2
        if c + 1 < NCHUNK:
            if c >= 1:
                pltpu.make_async_copy(
                    rows_v[1 - b], out_hbm.at[pl.ds(base + (c - 1) * CHUNK, CHUNK)],
                    ssem[1 - b]).wait()
            fire_gather(c + 1)
        pltpu.make_async_copy(
            table_hbm.at[idx_v.at[pl.ds(c * CHUNK, CHUNK)]],
            rows_v[b], gsem[b]).wait()
        pltpu.make_async_copy(
            rows_v[b], out_hbm.at[pl.ds(base + c * CHUNK, CHUNK)], ssem[b]).start()
    for c in (NCHUNK - 2, NCHUNK - 1):
        pltpu.make_async_copy(
            rows_v[c % 2], out_hbm.at[pl.ds(base + c * CHUNK, CHUNK)],
            ssem[c % 2]).wait()


@functools.cache
def _sc_gather():
    return pl.kernel(
        _sc_gather_body,
        mesh=plsc.VectorSubcoreMesh(core_axis_name="c", subcore_axis_name="s"),
        out_type=jax.ShapeDtypeStruct((ROWS_TOTAL, LEN, DIM), jnp.float32),
        scratch_types=[
            pltpu.VMEM((ROWS_PER_W,), jnp.int32),
            pltpu.VMEM((CHUNK, LEN, DIM), jnp.float32),
            pltpu.VMEM((CHUNK, LEN, DIM), jnp.float32),
            pltpu.SemaphoreType.DMA,
            pltpu.SemaphoreType.DMA,
            pltpu.SemaphoreType.DMA,
            pltpu.SemaphoreType.DMA,
        ],
        compiler_params=pltpu.CompilerParams(use_tc_tiling_on_sc=True),
    )


def kernel(query, prompt_keys, prompt_values):
    idx, loss = _tc_topk(query, prompt_keys)
    rows = _sc_gather()(idx.reshape(ROWS_TOTAL // 128, 128), prompt_values)
    return rows.reshape(BATCH, K, LEN, DIM), loss.reshape(())


# speedup vs baseline: 3.1149x; 1.0013x over previous
import functools

import jax
import jax.numpy as jnp
from jax import lax
from jax.experimental import pallas as pl
from jax.experimental.pallas import tpu as pltpu
from jax.experimental.pallas import tpu_sc as plsc

POOL = 512
K = 4
LEN = 8
DIM = 768
BATCH = 1024
ROW = LEN * DIM

NUM_WORKERS = 32
ROWS_TOTAL = BATCH * K
ROWS_PER_W = ROWS_TOTAL // NUM_WORKERS
CHUNK = 8
NCHUNK = ROWS_PER_W // CHUNK


def _topk_body(q_ref, k_ref, idx_ref, loss_ref):
    q = q_ref[...]
    k = k_ref[...]
    qn = q / jnp.maximum(jnp.sqrt(jnp.sum(q * q, axis=1, keepdims=True)), 1e-12)
    kn = k / jnp.maximum(jnp.sqrt(jnp.sum(k * k, axis=1, keepdims=True)), 1e-12)
    d = lax.dot_general(qn, kn, (((1,), (1,)), ((), ())),
                        preferred_element_type=jnp.float32)
    lane = lax.broadcasted_iota(jnp.int32, d.shape, 1)
    loss = jnp.float32(0.0)
    for t in range(K):
        m = jnp.max(d, axis=1, keepdims=True)
        im = jnp.min(jnp.where(d == m, lane, POOL), axis=1, keepdims=True)
        idx_ref[:, t : t + 1] = im
        loss = loss + jnp.sum(jnp.abs(m))
        d = jnp.where(lane == im, -jnp.inf, d)
    loss_ref[...] = jnp.full((1, 1), loss / jnp.float32(BATCH), jnp.float32)


def _tc_topk(query, keys_t):
    return pl.pallas_call(
        _topk_body,
        out_shape=(
            jax.ShapeDtypeStruct((BATCH, K), jnp.int32),
            jax.ShapeDtypeStruct((1, 1), jnp.float32),
        ),
    )(query, keys_t)


NBUF = 2


def _sc_gather_body(idx_hbm, table_hbm, out_hbm, idx_v, *bufs):
    wid = lax.axis_index("s") * 2 + lax.axis_index("c")
    base = wid * ROWS_PER_W
    rows_v = bufs[:NBUF]
    gsem = bufs[NBUF:2 * NBUF]
    ssem = bufs[2 * NBUF:]

    pltpu.sync_copy(idx_hbm.at[wid], idx_v)

    def fire_gather(c):
        pltpu.make_async_copy(
            table_hbm.at[idx_v.at[pl.ds(c * CHUNK, CHUNK)]],
            rows_v[c % NBUF], gsem[c % NBUF]).start()

    for c in range(NBUF - 1):
        fire_gather(c)
    for c in range(NCHUNK):
        b = c % NBUF
        if c + NBUF - 1 < NCHUNK:
            if c >= 1:
                bb = (c + NBUF - 1) % NBUF
                pltpu.make_async_copy(
                    rows_v[bb], out_hbm.at[pl.ds(base + (c - 1) * CHUNK, CHUNK)],
                    ssem[bb]).wait()
            fire_gather(c + NBUF - 1)
        pltpu.make_async_copy(
            table_hbm.at[idx_v.at[pl.ds(c * CHUNK, CHUNK)]],
            rows_v[b], gsem[b]).wait()
        pltpu.make_async_copy(
            rows_v[b], out_hbm.at[pl.ds(base + c * CHUNK, CHUNK)], ssem[b]).start()
    for c in range(max(NCHUNK - NBUF, 0), NCHUNK):
        pltpu.make_async_copy(
            rows_v[c % NBUF], out_hbm.at[pl.ds(base + c * CHUNK, CHUNK)],
            ssem[c % NBUF]).wait()


@functools.cache
def _sc_gather():
    return pl.kernel(
        _sc_gather_body,
        mesh=plsc.VectorSubcoreMesh(core_axis_name="c", subcore_axis_name="s"),
        out_type=jax.ShapeDtypeStruct((ROWS_TOTAL, LEN, DIM), jnp.float32),
        scratch_types=(
            [pltpu.VMEM((ROWS_PER_W,), jnp.int32)]
            + [pltpu.VMEM((CHUNK, LEN, DIM), jnp.float32)] * NBUF
            + [pltpu.SemaphoreType.DMA] * (2 * NBUF)
        ),
        compiler_params=pltpu.CompilerParams(use_tc_tiling_on_sc=True),
    )


def kernel(query, prompt_keys, prompt_values):
    idx, loss = _tc_topk(query, prompt_keys)
    rows = _sc_gather()(idx.reshape(ROWS_TOTAL // 128, 128), prompt_values)
    return rows.reshape(BATCH, K, LEN, DIM), loss.reshape(())


# speedup vs baseline: 3.6077x; 1.1582x over previous
import functools

import jax
import jax.numpy as jnp
from jax import lax
from jax.experimental import pallas as pl
from jax.experimental.pallas import tpu as pltpu
from jax.experimental.pallas import tpu_sc as plsc

POOL = 512
K = 4
LEN = 8
DIM = 768
BATCH = 1024

NUM_WORKERS = 32
ROWS_TOTAL = BATCH * K
TC_ROWS = 2048
SC_ROWS = ROWS_TOTAL - TC_ROWS
SC_PER_W = SC_ROWS // NUM_WORKERS
CHUNK = 8
NCHUNK = SC_PER_W // CHUNK
NBUF = 2
TC_BLOCK = 128
TC_GRID = TC_ROWS // TC_BLOCK


def _topk_body(q_ref, k_ref, idx_ref, loss_ref):
    q = q_ref[...]
    k = k_ref[...]
    qn = q / jnp.maximum(jnp.sqrt(jnp.sum(q * q, axis=1, keepdims=True)), 1e-12)
    kn = k / jnp.maximum(jnp.sqrt(jnp.sum(k * k, axis=1, keepdims=True)), 1e-12)
    d = lax.dot_general(qn, kn, (((1,), (1,)), ((), ())),
                        preferred_element_type=jnp.float32)
    lane = lax.broadcasted_iota(jnp.int32, d.shape, 1)
    loss = jnp.float32(0.0)
    for t in range(K):
        m = jnp.max(d, axis=1, keepdims=True)
        im = jnp.min(jnp.where(d == m, lane, POOL), axis=1, keepdims=True)
        idx_ref[:, t : t + 1] = im
        loss = loss + jnp.sum(jnp.abs(m))
        d = jnp.where(lane == im, -jnp.inf, d)
    loss_ref[...] = jnp.full((1, 1), loss / jnp.float32(BATCH), jnp.float32)


def _tc_topk(query, keys):
    return pl.pallas_call(
        _topk_body,
        out_shape=(
            jax.ShapeDtypeStruct((BATCH, K), jnp.int32),
            jax.ShapeDtypeStruct((1, 1), jnp.float32),
        ),
    )(query, keys)


def _sc_gather_body(idx_hbm, table_hbm, out_hbm, idx_v, *bufs):
    wid = lax.axis_index("s") * 2 + lax.axis_index("c")
    rows_v = bufs[:NBUF]
    gsem = bufs[NBUF:2 * NBUF]
    ssem = bufs[2 * NBUF:]

    pltpu.sync_copy(idx_hbm.at[TC_ROWS // 128 + wid // 2], idx_v)
    ioff = (wid % 2) * SC_PER_W
    base = TC_ROWS + wid * SC_PER_W

    def fire_gather(c):
        pltpu.make_async_copy(
            table_hbm.at[idx_v.at[pl.ds(ioff + c * CHUNK, CHUNK)]],
            rows_v[c % NBUF], gsem[c % NBUF]).start()

    for c in range(min(NBUF - 1, NCHUNK)):
        fire_gather(c)
    for c in range(NCHUNK):
        b = c % NBUF
        if c + NBUF - 1 < NCHUNK:
            if c >= 1:
                bb = (c + NBUF - 1) % NBUF
                pltpu.make_async_copy(
                    rows_v[bb], out_hbm.at[pl.ds(base + (c - 1) * CHUNK, CHUNK)],
                    ssem[bb]).wait()
            fire_gather(c + NBUF - 1)
        pltpu.make_async_copy(
            table_hbm.at[idx_v.at[pl.ds(ioff + c * CHUNK, CHUNK)]],
            rows_v[b], gsem[b]).wait()
        pltpu.make_async_copy(
            rows_v[b], out_hbm.at[pl.ds(base + c * CHUNK, CHUNK)], ssem[b]).start()
    for c in range(max(NCHUNK - NBUF, 0), NCHUNK):
        pltpu.make_async_copy(
            rows_v[c % NBUF], out_hbm.at[pl.ds(base + c * CHUNK, CHUNK)],
            ssem[c % NBUF]).wait()


@functools.cache
def _sc_gather():
    return pl.kernel(
        _sc_gather_body,
        mesh=plsc.VectorSubcoreMesh(core_axis_name="c", subcore_axis_name="s"),
        out_type=jax.ShapeDtypeStruct((ROWS_TOTAL, LEN, DIM), jnp.float32),
        scratch_types=(
            [pltpu.VMEM((128,), jnp.int32)]
            + [pltpu.VMEM((CHUNK, LEN, DIM), jnp.float32)] * NBUF
            + [pltpu.SemaphoreType.DMA] * (2 * NBUF)
        ),
        compiler_params=pltpu.CompilerParams(use_tc_tiling_on_sc=True),
    )


def _tc_fill_body(idx_ref, table_ref, part_ref, out_ref):
    del part_ref
    for i in range(TC_BLOCK):
        r = idx_ref[0, 0, i]
        out_ref[pl.ds(i, 1)] = table_ref[pl.ds(r, 1)]


def _tc_fill(idx32, table, part):
    idx32 = idx32.reshape(ROWS_TOTAL // 128, 1, 128)
    return pl.pallas_call(
        _tc_fill_body,
        grid=(TC_GRID,),
        in_specs=[
            pl.BlockSpec((1, 1, 128), lambda p: (p, 0, 0), memory_space=pltpu.SMEM),
            pl.BlockSpec((POOL, LEN, DIM), lambda p: (0, 0, 0)),
            pl.BlockSpec(memory_space=pl.ANY),
        ],
        out_specs=pl.BlockSpec((TC_BLOCK, LEN, DIM), lambda p: (p, 0, 0)),
        out_shape=jax.ShapeDtypeStruct((ROWS_TOTAL, LEN, DIM), jnp.float32),
        input_output_aliases={2: 0},
    )(idx32, table, part)


def kernel(query, prompt_keys, prompt_values):
    idx, loss = _tc_topk(query, prompt_keys)
    idx32 = idx.reshape(ROWS_TOTAL // 128, 128)
    part = _sc_gather()(idx32, prompt_values)
    rows = _tc_fill(idx32, prompt_values, part)
    return rows.reshape(BATCH, K, LEN, DIM), loss.reshape(())


# speedup vs baseline: 3.6684x; 1.0168x over previous
import functools

import jax
import jax.numpy as jnp
from jax import lax
from jax.experimental import pallas as pl
from jax.experimental.pallas import tpu as pltpu
from jax.experimental.pallas import tpu_sc as plsc

POOL = 512
K = 4
LEN = 8
DIM = 768
BATCH = 1024

NUM_WORKERS = 32
ROWS_TOTAL = BATCH * K
TC_ROWS = 2048
SC_ROWS = ROWS_TOTAL - TC_ROWS
SC_PER_W = SC_ROWS // NUM_WORKERS
CHUNK = 8
NCHUNK = SC_PER_W // CHUNK
NBUF = 2
TC_BLOCK = 128
TC_NBLK = TC_ROWS // TC_BLOCK


def _topk_body(q_ref, k_ref, idx_ref, loss_ref, canvas_ref):
    del canvas_ref
    q = q_ref[...]
    k = k_ref[...]
    qn = q / jnp.maximum(jnp.sqrt(jnp.sum(q * q, axis=1, keepdims=True)), 1e-12)
    kn = k / jnp.maximum(jnp.sqrt(jnp.sum(k * k, axis=1, keepdims=True)), 1e-12)
    d = lax.dot_general(qn, kn, (((1,), (1,)), ((), ())),
                        preferred_element_type=jnp.float32)
    lane = lax.broadcasted_iota(jnp.int32, d.shape, 1)
    loss = jnp.float32(0.0)
    for t in range(K):
        m = jnp.max(d, axis=1, keepdims=True)
        im = jnp.min(jnp.where(d == m, lane, POOL), axis=1, keepdims=True)
        idx_ref[:, t : t + 1] = im
        loss = loss + jnp.sum(jnp.abs(m))
        d = jnp.where(lane == im, -jnp.inf, d)
    loss_ref[...] = jnp.full((1, 1), loss / jnp.float32(BATCH), jnp.float32)


def _tc_topk(query, keys):
    return pl.pallas_call(
        _topk_body,
        out_shape=(
            jax.ShapeDtypeStruct((BATCH, K), jnp.int32),
            jax.ShapeDtypeStruct((1, 1), jnp.float32),
            jax.ShapeDtypeStruct((ROWS_TOTAL, LEN, DIM), jnp.float32),
        ),
        out_specs=(
            pl.BlockSpec((BATCH, K), lambda: (0, 0)),
            pl.BlockSpec((1, 1), lambda: (0, 0)),
            pl.BlockSpec(memory_space=pl.ANY),
        ),
    )(query, keys)


def _sc_gather_body(idx_hbm, table_hbm, canvas_hbm, dummy, idx_v, *bufs):
    wid = lax.axis_index("s") * 2 + lax.axis_index("c")
    rows_v = bufs[:NBUF]
    gsem = bufs[NBUF:2 * NBUF]
    ssem = bufs[2 * NBUF:]

    pltpu.sync_copy(idx_hbm.at[TC_ROWS // 128 + wid // 2], idx_v)
    ioff = (wid % 2) * SC_PER_W
    base = TC_ROWS + wid * SC_PER_W

    def fire_gather(c):
        pltpu.make_async_copy(
            table_hbm.at[idx_v.at[pl.ds(ioff + c * CHUNK, CHUNK)]],
            rows_v[c % NBUF], gsem[c % NBUF]).start()

    for c in range(min(NBUF - 1, NCHUNK)):
        fire_gather(c)
    for c in range(NCHUNK):
        b = c % NBUF
        if c + NBUF - 1 < NCHUNK:
            if c >= 1:
                bb = (c + NBUF - 1) % NBUF
                pltpu.make_async_copy(
                    rows_v[bb], canvas_hbm.at[pl.ds(base + (c - 1) * CHUNK, CHUNK)],
                    ssem[bb]).wait()
            fire_gather(c + NBUF - 1)
        pltpu.make_async_copy(
            table_hbm.at[idx_v.at[pl.ds(ioff + c * CHUNK, CHUNK)]],
            rows_v[b], gsem[b]).wait()
        pltpu.make_async_copy(
            rows_v[b], canvas_hbm.at[pl.ds(base + c * CHUNK, CHUNK)], ssem[b]).start()
    for c in range(max(NCHUNK - NBUF, 0), NCHUNK):
        pltpu.make_async_copy(
            rows_v[c % NBUF], canvas_hbm.at[pl.ds(base + c * CHUNK, CHUNK)],
            ssem[c % NBUF]).wait()
    pltpu.sync_copy(idx_v, dummy.at[wid])


@functools.cache
def _sc_gather():
    return pl.kernel(
        _sc_gather_body,
        mesh=plsc.VectorSubcoreMesh(core_axis_name="c", subcore_axis_name="s"),
        out_type=jax.ShapeDtypeStruct((NUM_WORKERS, 128), jnp.int32),
        scratch_types=(
            [pltpu.VMEM((128,), jnp.int32)]
            + [pltpu.VMEM((CHUNK, LEN, DIM), jnp.float32)] * NBUF
            + [pltpu.SemaphoreType.DMA] * (2 * NBUF)
        ),
        compiler_params=pltpu.CompilerParams(
            use_tc_tiling_on_sc=True, has_side_effects=True),
    )


def _tc_fill_body(idx_ref, table_ref, canvas_ref, dummy_ref, s0, s1, sem0, sem1):
    scr = (s0, s1)
    sem = (sem0, sem1)

    def copy_rows(blk, scratch):
        def body(i, _):
            r = idx_ref[0, blk, i]
            scratch[pl.ds(i, 1)] = table_ref[pl.ds(r, 1)]
            return 0
        lax.fori_loop(0, TC_BLOCK, body, 0)

    for blk in range(TC_NBLK):
        b = blk % 2
        if blk >= 2:
            pltpu.make_async_copy(
                scr[b], canvas_ref.at[pl.ds((blk - 2) * TC_BLOCK, TC_BLOCK)],
                sem[b]).wait()
        copy_rows(blk, scr[b])
        pltpu.make_async_copy(
            scr[b], canvas_ref.at[pl.ds(blk * TC_BLOCK, TC_BLOCK)], sem[b]).start()
    for blk in (TC_NBLK - 2, TC_NBLK - 1):
        pltpu.make_async_copy(
            scr[blk % 2], canvas_ref.at[pl.ds(blk * TC_BLOCK, TC_BLOCK)],
            sem[blk % 2]).wait()
    dummy_ref[...] = jnp.zeros((8, 128), jnp.float32)


def _tc_fill(idx3, table, canvas):
    return pl.pallas_call(
        _tc_fill_body,
        in_specs=[
            pl.BlockSpec((1, ROWS_TOTAL // 128, 128), lambda: (0, 0, 0),
                         memory_space=pltpu.SMEM),
            pl.BlockSpec((POOL, LEN, DIM), lambda: (0, 0, 0)),
            pl.BlockSpec(memory_space=pl.ANY),
        ],
        out_specs=pl.BlockSpec((8, 128), lambda: (0, 0)),
        out_shape=jax.ShapeDtypeStruct((8, 128), jnp.float32),
        scratch_shapes=[
            pltpu.VMEM((TC_BLOCK, LEN, DIM), jnp.float32),
            pltpu.VMEM((TC_BLOCK, LEN, DIM), jnp.float32),
            pltpu.SemaphoreType.DMA,
            pltpu.SemaphoreType.DMA,
        ],
        compiler_params=pltpu.CompilerParams(has_side_effects=True),
    )(idx3, table, canvas)


def kernel(query, prompt_keys, prompt_values):
    idx, loss, canvas = _tc_topk(query, prompt_keys)
    idx32 = idx.reshape(ROWS_TOTAL // 128, 128)
    d1 = _sc_gather()(idx32, prompt_values, canvas)
    d2 = _tc_fill(idx32.reshape(1, ROWS_TOTAL // 128, 128), prompt_values, canvas)
    rows, _, _ = lax.optimization_barrier((canvas, d1, d2))
    return rows.reshape(BATCH, K, LEN, DIM), loss.reshape(())
